# baseline (speedup 1.0000x reference)
import functools

import jax
import jax.numpy as jnp
from jax import lax
from jax.experimental import pallas as pl
from jax.experimental.pallas import tpu as pltpu
from jax.experimental.pallas import tpu_sc as plsc

NC = 2
NS = 16
NW = NC * NS
LANES = 16

BLK = 128


def _make_gather(BATCH, HIST, D, VP):
    b_per_w = BATCH // NW
    n_blk = b_per_w // BLK
    n_step = HIST * n_blk
    mesh = plsc.VectorSubcoreMesh(
        core_axis_name="c", subcore_axis_name="s", num_cores=NC, num_subcores=NS)

    @functools.partial(
        pl.kernel,
        out_type=jax.ShapeDtypeStruct((HIST, D, BATCH), jnp.float32),
        mesh=mesh,
        scratch_types=[
            pltpu.VMEM((HIST, b_per_w), jnp.int32),
            pltpu.VMEM((BLK,), jnp.int32),
            pltpu.VMEM((BLK,), jnp.int32),
            pltpu.VMEM((BLK,), jnp.int32),
            pltpu.VMEM((BLK,), jnp.int32),
            pltpu.VMEM((BLK, 128), jnp.float32),
            pltpu.VMEM((BLK, 128), jnp.float32),
            pltpu.VMEM((D, BLK), jnp.float32),
            pltpu.VMEM((D, BLK), jnp.float32),
            pltpu.SemaphoreType.DMA,
            pltpu.SemaphoreType.DMA,
            pltpu.SemaphoreType.DMA,
            pltpu.SemaphoreType.DMA,
        ],
        compiler_params=pltpu.CompilerParams(needs_layout_passes=False),
    )
    def gather_kernel(idxT_hbm, packed_hbm, out_hbm,
                      slab, gia, gib, qa, qb, ra, rb, oa, ob,
                      gsa, gsb, osa, osb):
        wid = lax.axis_index("s") * NC + lax.axis_index("c")
        base = wid * b_per_w
        iota = lax.iota(jnp.int32, LANES)

        pltpu.sync_copy(idxT_hbm.at[:, pl.ds(base, b_per_w)], slab)

        def build(gi, q, t):
            h = t >> 2
            j = t & 3
            for k in range(BLK // LANES):
                v = slab[h, pl.ds(j * BLK + k * LANES, LANES)]
                gi[pl.ds(k * LANES, LANES)] = lax.shift_right_logical(v, 2)
                q[pl.ds(k * LANES, LANES)] = lax.shift_left(v & 3, 5)

        def fire(gi, r, sem):
            return pltpu.async_copy(packed_hbm.at[gi], r, sem)

        def compact(r, q, o):
            for b0 in range(BLK // LANES):
                rows = b0 * LANES + iota
                qv = q[pl.ds(b0 * LANES, LANES)]
                for d in range(D):
                    vals = plsc.load_gather(r, [rows, qv + d])
                    o[d, pl.ds(b0 * LANES, LANES)] = vals

        def store(o, t, sem):
            h = t >> 2
            j = t & 3
            return pltpu.async_copy(
                o, out_hbm.at[h, :, pl.ds(base + j * BLK, BLK)], sem)

        build(gia, qa, 0)
        fire(gia, ra, gsa)

        def body(s):
            t0 = 2 * s
            build(gib, qb, t0 + 1)
            fire(gib, rb, gsb)

            @pl.when(s > 0)
            def _():
                pltpu.make_async_copy(oa, out_hbm.at[0, :, pl.ds(0, BLK)], osa).wait()
            pltpu.make_async_copy(packed_hbm.at[gia], ra, gsa).wait()
            compact(ra, qa, oa)
            store(oa, t0, osa)

            @pl.when(t0 + 2 < n_step)
            def _():
                build(gia, qa, t0 + 2)
                fire(gia, ra, gsa)

            @pl.when(s > 0)
            def _():
                pltpu.make_async_copy(ob, out_hbm.at[0, :, pl.ds(0, BLK)], osb).wait()
            pltpu.make_async_copy(packed_hbm.at[gib], rb, gsb).wait()
            compact(rb, qb, ob)
            store(ob, t0 + 1, osb)

        pl.loop(0, n_step // 2)(body)
        pltpu.make_async_copy(oa, out_hbm.at[0, :, pl.ds(0, BLK)], osa).wait()
        pltpu.make_async_copy(ob, out_hbm.at[0, :, pl.ds(0, BLK)], osb).wait()

    return gather_kernel


def kernel(inputs, table):
    BATCH, HIST = inputs.shape
    V, D = table.shape
    idxT = jnp.transpose(inputs.astype(jnp.int32))
    packed = jnp.reshape(table, (V * D // 128, 128))
    outT = _make_gather(BATCH, HIST, D, V * D // 128)(idxT, packed)
    return jnp.transpose(outT, (2, 0, 1))
